# baseline (speedup 1.0000x reference)
import functools

import jax
import jax.numpy as jnp
from jax import lax
from jax.experimental import pallas as pl
from jax.experimental.pallas import tpu as pltpu
from jax.experimental.pallas import tpu_sc as plsc

BATCH = 16384
D = 20
ROWS = 9

_info = plsc.get_sparse_core_info()
_NC, _NS, _L = _info.num_cores, _info.num_subcores, _info.num_lanes
_NW = _NC * _NS
_BPW = BATCH // _NW
_GROUPS = _BPW // _L

_mesh = plsc.VectorSubcoreMesh(core_axis_name="c", subcore_axis_name="s")


@functools.partial(
    pl.kernel,
    mesh=_mesh,
    out_type=jax.ShapeDtypeStruct((D, BATCH), jnp.float32),
    scratch_types=[
        pltpu.VMEM((_BPW,), jnp.int32),
        pltpu.VMEM((ROWS, D), jnp.float32),
        pltpu.VMEM((D, _BPW), jnp.float32),
        pltpu.SemaphoreType.DMA,
        pltpu.SemaphoreType.DMA,
    ],
    compiler_params=pltpu.CompilerParams(
        needs_layout_passes=False, skip_device_barrier=True
    ),
)
def _sc_lookup(lengths_hbm, table_hbm, out_hbm, len_v, tab_v, out_v, sem1, sem2):
    wid = lax.axis_index("s") * _NC + lax.axis_index("c")
    base = wid * _BPW
    len_cp = pltpu.async_copy(lengths_hbm.at[pl.ds(base, _BPW)], len_v, sem1)
    tab_cp = pltpu.async_copy(table_hbm, tab_v, sem2)
    iota = lax.iota(jnp.int32, _L)

    tab_cp.wait()
    iota_c = jnp.minimum(iota, jnp.full((_L,), ROWS - 1, jnp.int32))
    cols = [
        plsc.load_gather(tab_v, [iota_c, jnp.full((_L,), d, jnp.int32)])
        for d in range(D)
    ]
    len_cp.wait()

    @plsc.parallel_loop(0, _GROUPS, 1, unroll=1)
    def group(g):
        l = len_v[pl.ds(g * _L, _L)]
        lm1 = l - jnp.full((_L,), 1, jnp.int32)
        lo = jnp.maximum(lm1, jnp.zeros((_L,), jnp.int32))
        ebits = lax.shift_right_logical(
            lax.bitcast_convert_type(lm1.astype(jnp.float32), jnp.int32),
            jnp.full((_L,), 23, jnp.int32),
        )
        hi = jnp.minimum(
            ebits - jnp.full((_L,), 125, jnp.int32),
            jnp.full((_L,), 8, jnp.int32),
        )
        idx = jnp.where(l > jnp.full((_L,), 4, jnp.int32), hi, lo)
        for d in range(D):
            out_v[d, pl.ds(g * _L, _L)] = jnp.take_along_axis(
                cols[d], idx, axis=0, mode="promise_in_bounds"
            )

    pltpu.sync_copy(out_v, out_hbm.at[:, pl.ds(base, _BPW)])


def kernel(lengths, table):
    return _sc_lookup(lengths, table).T


# speedup vs baseline: 1.0055x; 1.0055x over previous
import functools

import jax
import jax.numpy as jnp
from jax import lax
from jax.experimental import pallas as pl
from jax.experimental.pallas import tpu as pltpu
from jax.experimental.pallas import tpu_sc as plsc

BATCH = 16384
D = 20
ROWS = 9

_info = plsc.get_sparse_core_info()
_NC, _NS, _L = _info.num_cores, _info.num_subcores, _info.num_lanes
_NW = _NC * _NS
_BPW = BATCH // _NW
_GROUPS = _BPW // _L

_mesh = plsc.VectorSubcoreMesh(core_axis_name="c", subcore_axis_name="s")


@functools.partial(
    pl.kernel,
    mesh=_mesh,
    out_type=jax.ShapeDtypeStruct((D, BATCH), jnp.float32),
    scratch_types=[
        pltpu.VMEM((_BPW,), jnp.int32),
        pltpu.VMEM((ROWS, D), jnp.float32),
        pltpu.VMEM((D, _BPW), jnp.float32),
        pltpu.SemaphoreType.DMA,
        pltpu.SemaphoreType.DMA,
    ],
    compiler_params=pltpu.CompilerParams(
        needs_layout_passes=False, skip_device_barrier=True
    ),
)
def _sc_lookup(lengths_hbm, table_hbm, out_hbm, len_v, tab_v, out_v, sem1, sem2):
    wid = lax.axis_index("s") * _NC + lax.axis_index("c")
    base = wid * _BPW
    len_cp = pltpu.async_copy(lengths_hbm.at[pl.ds(base, _BPW)], len_v, sem1)
    tab_cp = pltpu.async_copy(table_hbm, tab_v, sem2)
    iota = lax.iota(jnp.int32, _L)

    tab_cp.wait()
    iota_c = jnp.minimum(iota, jnp.full((_L,), ROWS - 1, jnp.int32))
    cols = [
        plsc.load_gather(tab_v, [iota_c, jnp.full((_L,), d, jnp.int32)])
        for d in range(D)
    ]
    len_cp.wait()

    @plsc.parallel_loop(0, _GROUPS, 1, unroll=1)
    def group(g):
        l = len_v[pl.ds(g * _L, _L)]
        lm1 = l - jnp.full((_L,), 1, jnp.int32)
        lo = jnp.minimum(
            jnp.maximum(lm1, jnp.zeros((_L,), jnp.int32)),
            jnp.full((_L,), 3, jnp.int32),
        )
        ebits = lax.shift_right_logical(
            lax.bitcast_convert_type(lm1.astype(jnp.float32), jnp.int32),
            jnp.full((_L,), 23, jnp.int32),
        )
        hi = jnp.minimum(
            ebits - jnp.full((_L,), 125, jnp.int32),
            jnp.full((_L,), 8, jnp.int32),
        )
        idx = jnp.where(l > jnp.full((_L,), 4, jnp.int32), hi, lo)
        for d in range(D):
            out_v[d, pl.ds(g * _L, _L)] = jnp.take_along_axis(
                cols[d], idx, axis=0, mode="promise_in_bounds"
            )

    pltpu.sync_copy(out_v, out_hbm.at[:, pl.ds(base, _BPW)])


def kernel(lengths, table):
    return _sc_lookup(lengths, table).T
